# Initial kernel scaffold; baseline (speedup 1.0000x reference)
import jax
import jax.numpy as jnp
from jax.experimental import pallas as pl
from jax.experimental.pallas import tpu as pltpu
from jax.experimental.pallas import tpu_sc as plsc

EMBEDDING_DIM = 64
WINDOW = 128


def kernel(x, weight):
    batch, seq = x.shape
    n = batch * seq
    idx = x.reshape(1, n)

    mesh = plsc.VectorSubcoreMesh(core_axis_name="core", subcore_axis_name="subcore")

    @pl.kernel(
        out_type=jax.ShapeDtypeStruct((n, EMBEDDING_DIM), weight.dtype),
        mesh=mesh,
    )
    def gather_kernel(w_hbm, i_hbm, o_hbm):
        def body(i_vmem, o_vmem):
            pltpu.sync_copy(w_hbm.at[i_vmem.at[0]], o_vmem)

        pltpu.emit_pipeline(
            body,
            grid=(n // WINDOW,),
            in_specs=[pl.BlockSpec((1, WINDOW), index_map=lambda i: (0, i))],
            out_specs=[pl.BlockSpec((WINDOW, EMBEDDING_DIM), index_map=lambda i: (i, 0))],
            core_axis_name=("core", "subcore"),
            dimension_semantics=(pltpu.PARALLEL,),
        )(i_hbm, o_hbm)

    out = gather_kernel(weight, idx)
    return out.reshape(batch, seq, EMBEDDING_DIM)


# speedup vs baseline: 4.5662x; 4.5662x over previous
import functools

import jax
import jax.numpy as jnp
from jax import lax
from jax.experimental import pallas as pl
from jax.experimental.pallas import tpu as pltpu
from jax.experimental.pallas import tpu_sc as plsc

EMBEDDING_DIM = 64
NUM_CORES = 2
NUM_SUBCORES = 16
NUM_WORKERS = NUM_CORES * NUM_SUBCORES
CHUNK = 800


def kernel(x, weight):
    batch, seq = x.shape
    n = batch * seq
    idx = x.reshape(n)
    per_worker = n // NUM_WORKERS
    n_chunks = per_worker // CHUNK

    mesh = plsc.VectorSubcoreMesh(core_axis_name="c", subcore_axis_name="s")

    @functools.partial(
        pl.kernel,
        mesh=mesh,
        compiler_params=pltpu.CompilerParams(use_tc_tiling_on_sc=False),
        out_type=jax.ShapeDtypeStruct((n, EMBEDDING_DIM), weight.dtype),
        scratch_types=[
            pltpu.VMEM((CHUNK,), jnp.int32),
            pltpu.VMEM((CHUNK, EMBEDDING_DIM), jnp.float32),
            pltpu.SemaphoreType.DMA,
        ],
    )
    def gather_k(table_hbm, idx_hbm, out_hbm, idx_v, rows_v, sem):
        wid = lax.axis_index("s") * NUM_CORES + lax.axis_index("c")
        base = wid * per_worker

        @pl.loop(0, n_chunks)
        def _(c):
            off = base + c * CHUNK
            pltpu.sync_copy(idx_hbm.at[pl.ds(off, CHUNK)], idx_v)
            pltpu.async_copy(table_hbm.at[idx_v], rows_v, sem).wait()
            pltpu.sync_copy(rows_v, out_hbm.at[pl.ds(off, CHUNK)])

    out = gather_k(weight, idx)
    return out.reshape(batch, seq, EMBEDDING_DIM)


# speedup vs baseline: 4.6144x; 1.0106x over previous
import functools

import jax
import jax.numpy as jnp
from jax import lax
from jax.experimental import pallas as pl
from jax.experimental.pallas import tpu as pltpu
from jax.experimental.pallas import tpu_sc as plsc

EMBEDDING_DIM = 64
NUM_CORES = 2
NUM_SUBCORES = 16
NUM_WORKERS = NUM_CORES * NUM_SUBCORES
CHUNK = 800


def kernel(x, weight):
    batch, seq = x.shape
    n = batch * seq
    idx = x.reshape(n)
    per_worker = n // NUM_WORKERS
    n_chunks = per_worker // CHUNK

    mesh = plsc.VectorSubcoreMesh(core_axis_name="c", subcore_axis_name="s")

    @functools.partial(
        pl.kernel,
        mesh=mesh,
        compiler_params=pltpu.CompilerParams(use_tc_tiling_on_sc=False),
        out_type=jax.ShapeDtypeStruct((n, EMBEDDING_DIM), weight.dtype),
        scratch_types=[
            pltpu.VMEM((per_worker,), jnp.int32),
            pltpu.VMEM((CHUNK, EMBEDDING_DIM), jnp.float32),
            pltpu.VMEM((CHUNK, EMBEDDING_DIM), jnp.float32),
            pltpu.SemaphoreType.DMA,
            pltpu.SemaphoreType.DMA,
        ],
    )
    def gather_k(table_hbm, idx_hbm, out_hbm, idx_v, rows_a, rows_b, sem_g, sem_w):
        wid = lax.axis_index("s") * NUM_CORES + lax.axis_index("c")
        base = wid * per_worker
        pltpu.sync_copy(idx_hbm.at[pl.ds(base, per_worker)], idx_v)

        bufs = (rows_a, rows_b)
        gh = [None, None]
        wr = [None, None]
        gh[0] = pltpu.async_copy(
            table_hbm.at[idx_v.at[pl.ds(0, CHUNK)]], bufs[0], sem_g
        )
        for c in range(n_chunks):
            b = c % 2
            gh[b].wait()
            if c + 1 < n_chunks:
                nb = (c + 1) % 2
                if wr[nb] is not None:
                    wr[nb].wait()
                gh[nb] = pltpu.async_copy(
                    table_hbm.at[idx_v.at[pl.ds((c + 1) * CHUNK, CHUNK)]],
                    bufs[nb],
                    sem_g,
                )
            wr[b] = pltpu.async_copy(
                bufs[b], out_hbm.at[pl.ds(base + c * CHUNK, CHUNK)], sem_w
            )
        for w in wr:
            if w is not None:
                w.wait()

    out = gather_k(weight, idx)
    return out.reshape(batch, seq, EMBEDDING_DIM)


# speedup vs baseline: 4.6687x; 1.0118x over previous
import functools

import jax
import jax.numpy as jnp
from jax import lax
from jax.experimental import pallas as pl
from jax.experimental.pallas import tpu as pltpu
from jax.experimental.pallas import tpu_sc as plsc

EMBEDDING_DIM = 64
NUM_CORES = 2
NUM_SUBCORES = 16
NUM_WORKERS = NUM_CORES * NUM_SUBCORES
NBUF = 4
CHUNK = 400


def kernel(x, weight):
    batch, seq = x.shape
    n = batch * seq
    idx = x.reshape(n)
    per_worker = n // NUM_WORKERS
    n_chunks = per_worker // CHUNK

    mesh = plsc.VectorSubcoreMesh(core_axis_name="c", subcore_axis_name="s")

    @functools.partial(
        pl.kernel,
        mesh=mesh,
        compiler_params=pltpu.CompilerParams(use_tc_tiling_on_sc=False),
        out_type=jax.ShapeDtypeStruct((n, EMBEDDING_DIM), weight.dtype),
        scratch_types=[
            pltpu.VMEM((per_worker,), jnp.int32),
        ]
        + [pltpu.VMEM((CHUNK, EMBEDDING_DIM), jnp.float32) for _ in range(NBUF)]
        + [pltpu.SemaphoreType.DMA for _ in range(2 * NBUF)],
    )
    def gather_k(table_hbm, idx_hbm, out_hbm, idx_v, *scratch):
        bufs = scratch[:NBUF]
        gsems = scratch[NBUF : 2 * NBUF]
        wsems = scratch[2 * NBUF :]
        wid = lax.axis_index("s") * NUM_CORES + lax.axis_index("c")
        base = wid * per_worker
        pltpu.sync_copy(idx_hbm.at[pl.ds(base, per_worker)], idx_v)

        def start_gather(c):
            b = c % NBUF
            return pltpu.async_copy(
                table_hbm.at[idx_v.at[pl.ds(c * CHUNK, CHUNK)]], bufs[b], gsems[b]
            )

        gh = [None] * NBUF
        wr = [None] * NBUF
        for c in range(NBUF - 1):
            gh[c % NBUF] = start_gather(c)
        for c in range(n_chunks):
            b = c % NBUF
            nxt = c + NBUF - 1
            if nxt < n_chunks:
                nb = nxt % NBUF
                if wr[nb] is not None:
                    wr[nb].wait()
                gh[nb] = start_gather(nxt)
            gh[b].wait()
            wr[b] = pltpu.async_copy(
                bufs[b], out_hbm.at[pl.ds(base + c * CHUNK, CHUNK)], wsems[b]
            )
        for w in wr:
            if w is not None:
                w.wait()

    out = gather_k(weight, idx)
    return out.reshape(batch, seq, EMBEDDING_DIM)
